# Initial kernel scaffold; baseline (speedup 1.0000x reference)
import jax
import jax.numpy as jnp
from jax.experimental import pallas as pl


def kernel(mask, centers, log_scales, rho_real, rho_imag):
    raise NotImplementedError("write your pallas kernel here")


# speedup vs baseline: 22.2988x; 22.2988x over previous
import functools

import jax
import jax.numpy as jnp
from jax.experimental import pallas as pl
from jax.experimental.pallas import tpu as pltpu

_NZ, _NX, _NY = 128, 128, 128
_R = 3
_W = 2 * _R + 1
_SIGMA_SQ = 9.0


def _vox_kernel(packed_ref, az_ref, cz_ref, ax_ref, cx_ref, ay_ref, by_ref,
                rr_ref, ri_ref, outr_ref, outi_ref):
    @pl.when(pl.program_id(0) == 0)
    def _init():
        outr_ref[...] = jnp.zeros_like(outr_ref)
        outi_ref[...] = jnp.zeros_like(outi_ref)

    kz = jax.lax.broadcasted_iota(jnp.int32, (_W, _W, _NY), 0).astype(jnp.float32)
    kx = jax.lax.broadcasted_iota(jnp.int32, (_W, _W, _NY), 1).astype(jnp.float32)
    vy = jax.lax.broadcasted_iota(jnp.int32, (_W, _W, _NY), 2).astype(jnp.float32)

    m_total = packed_ref.shape[0]

    def body(m, _):
        p = packed_ref[m]
        cvy = p % 128
        p = p // 128
        dx = p % 8
        p = p // 8
        dz = p % 8
        p = p // 8
        x0 = p % 128
        z0 = p // 128

        uz = kz * az_ref[m] + cz_ref[m]
        ux = kx * ax_ref[m] + cx_ref[m]
        uy = vy * ay_ref[m] + by_ref[m]
        dist_sq = uz * uz + ux * ux + uy * uy

        valid = (dist_sq <= _SIGMA_SQ)
        valid &= jnp.abs(kz - dz.astype(jnp.float32)) <= 3.0
        valid &= jnp.abs(kx - dx.astype(jnp.float32)) <= 3.0
        valid &= jnp.abs(vy - cvy.astype(jnp.float32)) <= 3.0
        w = jnp.where(valid, jnp.exp(-0.5 * dist_sq), 0.0)

        outr_ref[pl.ds(z0, _W), pl.ds(x0, _W), :] += rr_ref[m] * w
        outi_ref[pl.ds(z0, _W), pl.ds(x0, _W), :] += ri_ref[m] * w
        return 0

    jax.lax.fori_loop(0, m_total, body, 0)


@functools.partial(jax.jit, static_argnames=())
def _voxelize(centers, log_scales, rho_real, rho_imag):
    a = 2.0 / (_NZ - 1.0)
    gcf = (centers + 1.0) * 0.5 * (_NZ - 1.0)
    cv = jnp.round(gcf).astype(jnp.int32)
    z0 = jnp.clip(cv[:, 0] - _R, 0, _NZ - _W)
    x0 = jnp.clip(cv[:, 1] - _R, 0, _NX - _W)

    inv_s = 1.0 / (jnp.exp(log_scales) + 1e-08)
    az = a * inv_s[:, 0]
    ax = a * inv_s[:, 1]
    ay = a * inv_s[:, 2]
    cz = (z0.astype(jnp.float32) * a - 1.0 - centers[:, 0]) * inv_s[:, 0]
    cx = (x0.astype(jnp.float32) * a - 1.0 - centers[:, 1]) * inv_s[:, 1]
    by = (-1.0 - centers[:, 2]) * inv_s[:, 2]

    packed = ((((z0 * 128 + x0) * 8 + (cv[:, 0] - z0)) * 8
               + (cv[:, 1] - x0)) * 128 + cv[:, 2]).astype(jnp.int32)

    m = packed.shape[0]
    chunk = 2048
    n_chunks = (m + chunk - 1) // chunk
    pad = n_chunks * chunk - m
    if pad:
        packed = jnp.pad(packed, (0, pad))
        az, cz, ax, cx, ay, by = (jnp.pad(v, (0, pad))
                                  for v in (az, cz, ax, cx, ay, by))
        rho_real = jnp.pad(rho_real, (0, pad))
        rho_imag = jnp.pad(rho_imag, (0, pad))
    smem_spec = pl.BlockSpec((chunk,), lambda i: (i,),
                             memory_space=pltpu.SMEM)
    out_spec = pl.BlockSpec((_NZ, _NX, _NY), lambda i: (0, 0, 0),
                            memory_space=pltpu.VMEM)
    outr, outi = pl.pallas_call(
        _vox_kernel,
        grid=(n_chunks,),
        out_shape=[jax.ShapeDtypeStruct((_NZ, _NX, _NY), jnp.float32)] * 2,
        in_specs=[smem_spec] * 9,
        out_specs=[out_spec] * 2,
    )(packed, az, cz, ax, cx, ay, by, rho_real, rho_imag)
    return outr, outi


def kernel(mask, centers, log_scales, rho_real, rho_imag):
    vol_r, vol_i = _voxelize(centers, log_scales, rho_real, rho_imag)
    volume = vol_r + 1j * vol_i
    kspace = jnp.fft.fftshift(jnp.fft.fftn(jnp.fft.ifftshift(volume),
                                           norm='ortho'))
    return (volume, mask * kspace)
